# baseline (device time: 12191 ns/iter reference)
import jax
import jax.numpy as jnp
from jax import lax
from jax.experimental import pallas as pl
from jax.experimental.pallas import tpu as pltpu

K = 4


def kernel(x):
    _, M, N2 = x.shape
    N = N2 // 2
    H = M // 2
    C = H // K

    def body(x_ref, out_ref, xsend, xrecv, ysend,
             xs_sems, xr_sems, ys_sems, yr_sems):
        my_x = lax.axis_index("x")
        my_y = lax.axis_index("y")
        xpeer = (1 - my_x, my_y)
        ypeer = (my_x, 1 - my_y)
        row0 = my_y * H
        other0 = (1 - my_y) * H

        barrier_sem = pltpu.get_barrier_semaphore()
        for nbr in (xpeer, ypeer):
            pl.semaphore_signal(
                barrier_sem, inc=1, device_id=nbr,
                device_id_type=pl.DeviceIdType.MESH,
            )
        pl.semaphore_wait(barrier_sem, 2)

        xsend[...] = x_ref[0, pl.ds(row0, H), pl.ds((1 - my_x) * N, N)].astype(
            jnp.bfloat16
        )
        xrd = []
        for k in range(K):
            r = pltpu.make_async_remote_copy(
                src_ref=xsend.at[pl.ds(k * C, C)],
                dst_ref=xrecv.at[pl.ds(k * C, C)],
                send_sem=xs_sems.at[k],
                recv_sem=xr_sems.at[k],
                device_id=xpeer,
                device_id_type=pl.DeviceIdType.MESH,
            )
            r.start()
            xrd.append(r)

        yrd = []
        for k in range(K):
            xrd[k].wait_recv()
            ysend[pl.ds(k * C, C)] = (
                x_ref[0, pl.ds(row0 + k * C, C), pl.ds(my_x * N, N)].astype(
                    jnp.bfloat16
                )
                + xrecv[pl.ds(k * C, C)]
            )
            r = pltpu.make_async_remote_copy(
                src_ref=ysend.at[pl.ds(k * C, C)],
                dst_ref=out_ref.at[pl.ds(row0 + k * C, C)],
                send_sem=ys_sems.at[k],
                recv_sem=yr_sems.at[k],
                device_id=ypeer,
                device_id_type=pl.DeviceIdType.MESH,
            )
            r.start()
            yrd.append(r)
            out_ref[pl.ds(row0 + k * C, C), :] = ysend[pl.ds(k * C, C)]

        for k in range(K):
            yrd[k].wait_recv()
        for k in range(K):
            xrd[k].wait_send()
            yrd[k].wait_send()

    return pl.pallas_call(
        body,
        out_shape=jax.ShapeDtypeStruct((M, N), jnp.bfloat16),
        in_specs=[pl.BlockSpec(memory_space=pltpu.VMEM)],
        out_specs=pl.BlockSpec(memory_space=pltpu.VMEM),
        scratch_shapes=[
            pltpu.VMEM((H, N), jnp.bfloat16),
            pltpu.VMEM((H, N), jnp.bfloat16),
            pltpu.VMEM((H, N), jnp.bfloat16),
            pltpu.SemaphoreType.DMA((K,)),
            pltpu.SemaphoreType.DMA((K,)),
            pltpu.SemaphoreType.DMA((K,)),
            pltpu.SemaphoreType.DMA((K,)),
        ],
        compiler_params=pltpu.CompilerParams(collective_id=0),
    )(x)


# device time: 12115 ns/iter; 1.0063x vs baseline; 1.0063x over previous
import jax
import jax.numpy as jnp
from jax import lax
from jax.experimental import pallas as pl
from jax.experimental.pallas import tpu as pltpu

K = 4


def kernel(x):
    _, M, N2 = x.shape
    N = N2 // 2
    H = M // 2
    C = H // K

    def body(x_ref, out_ref, xsend, xrecv, yrecv,
             xs_sems, xr_sems, ys_sems, yr_sems):
        my_x = lax.axis_index("x")
        my_y = lax.axis_index("y")
        xpeer = (1 - my_x, my_y)
        ypeer = (my_x, 1 - my_y)
        row0 = my_y * H
        other0 = (1 - my_y) * H

        barrier_sem = pltpu.get_barrier_semaphore()
        for nbr in (xpeer, ypeer):
            pl.semaphore_signal(
                barrier_sem, inc=1, device_id=nbr,
                device_id_type=pl.DeviceIdType.MESH,
            )
        xsend[...] = x_ref[0, pl.ds(row0, H), pl.ds((1 - my_x) * N, N)].astype(
            jnp.bfloat16
        )
        pl.semaphore_wait(barrier_sem, 2)

        xrd = []
        for k in range(K):
            r = pltpu.make_async_remote_copy(
                src_ref=xsend.at[pl.ds(k * C, C)],
                dst_ref=xrecv.at[pl.ds(k * C, C)],
                send_sem=xs_sems.at[k],
                recv_sem=xr_sems.at[k],
                device_id=xpeer,
                device_id_type=pl.DeviceIdType.MESH,
            )
            r.start()
            xrd.append(r)

        yrd = []
        for k in range(K):
            xrd[k].wait_recv()
            r = pltpu.make_async_remote_copy(
                src_ref=xrecv.at[pl.ds(k * C, C)],
                dst_ref=yrecv.at[pl.ds(k * C, C)],
                send_sem=ys_sems.at[k],
                recv_sem=yr_sems.at[k],
                device_id=ypeer,
                device_id_type=pl.DeviceIdType.MESH,
            )
            r.start()
            yrd.append(r)

        out_ref[pl.ds(row0, H), :] = (
            x_ref[0, pl.ds(row0, H), pl.ds(my_x * N, N)].astype(jnp.bfloat16)
            + xrecv[...]
        )

        for k in range(K):
            yrd[k].wait_recv()
            out_ref[pl.ds(other0 + k * C, C), :] = (
                x_ref[0, pl.ds(other0 + k * C, C), pl.ds(my_x * N, N)].astype(
                    jnp.bfloat16
                )
                + yrecv[pl.ds(k * C, C)]
            )

        for k in range(K):
            xrd[k].wait_send()
            yrd[k].wait_send()

    return pl.pallas_call(
        body,
        out_shape=jax.ShapeDtypeStruct((M, N), jnp.bfloat16),
        in_specs=[pl.BlockSpec(memory_space=pltpu.VMEM)],
        out_specs=pl.BlockSpec(memory_space=pltpu.VMEM),
        scratch_shapes=[
            pltpu.VMEM((H, N), jnp.bfloat16),
            pltpu.VMEM((H, N), jnp.bfloat16),
            pltpu.VMEM((H, N), jnp.bfloat16),
            pltpu.SemaphoreType.DMA((K,)),
            pltpu.SemaphoreType.DMA((K,)),
            pltpu.SemaphoreType.DMA((K,)),
            pltpu.SemaphoreType.DMA((K,)),
        ],
        compiler_params=pltpu.CompilerParams(collective_id=0),
    )(x)
